# baseline (speedup 1.0000x reference)
import jax
import jax.numpy as jnp
from jax.experimental import pallas as pl
from jax.experimental.pallas import tpu as pltpu

_TEMP = 0.2
_IOU_TH = 0.5
_LOG2E = 1.4426950408889634
_LN2 = 0.6931471805599453
_P = 2048
_N0 = 2048
_N1 = 2048
_K = 128
_M = _P + _N0 + _N1
_R = _P + _N0
_BR = 256
_CC = 512
_NB = _R // _BR
_SEG = _P // _CC
_KEEP_LANE = 96
_KA = 136


def _loss_kernel(oh_ref, prow_ref, nrow_ref,
                 pbf_ref, n0bf_ref, n1bf_ref, out_ref,
                 acc_ref, s_ref, n_ref):
    rb = pl.program_id(0)
    is_pos = rb < _NB // 2

    @pl.when(rb == 0)
    def _():
        lane = jax.lax.broadcasted_iota(jnp.int32, (1, 128), 1)
        sub = jax.lax.broadcasted_iota(jnp.int32, (128, 1), 0)
        s_acc = jnp.zeros((128, 128), jnp.float32)
        for seg, f in ((0, pbf_ref), (1, n0bf_ref), (2, n1bf_ref)):
            s_acc = s_acc + jax.lax.dot_general(
                oh_ref[seg * _P:(seg + 1) * _P, :], f[:, 0:_K],
                (((0,), (0,)), ((), ())), preferred_element_type=jnp.float32)
        s_ref[...] = jnp.where(sub == _KEEP_LANE, 0.0,
                               s_acc).astype(jnp.bfloat16)
        n_ref[...] = jnp.where(
            lane == _KEEP_LANE, 0.0,
            jnp.sum(oh_ref[...].astype(jnp.float32), axis=0, keepdims=True))
    rows = jnp.where(is_pos, prow_ref[...], nrow_ref[...])
    rows_bf = (rows * (_LOG2E / _TEMP)).astype(jnp.bfloat16)
    rows_bf32 = rows_bf.astype(jnp.float32)

    srcs = [pbf_ref] * _SEG + [n0bf_ref] * _SEG + [n1bf_ref] * _SEG

    def chunks():
        for c in range(_M // _CC):
            lo = (c % _SEG) * _CC
            yield srcs[c][lo:lo + _CC, :]

    rows_p1 = jnp.pad(rows_bf, ((0, 0), (0, _KA - _K)))
    m = None
    for chunk in chunks():
        sim = jax.lax.dot_general(
            rows_p1, chunk, (((1,), (1,)), ((), ())),
            preferred_element_type=jnp.float32)
        cm = jnp.max(sim, axis=1, keepdims=True)
        m = cm if m is None else jnp.maximum(m, cm)

    nm_b = (-m).astype(jnp.bfloat16)
    nm_b32 = nm_b.astype(jnp.float32)
    lane_a = jax.lax.broadcasted_iota(jnp.int32, (1, _KA), 1)
    rows_p2 = jnp.where(lane_a == _K, nm_b, rows_p1)
    s = None
    for chunk in chunks():
        sim2 = jax.lax.dot_general(
            rows_p2, chunk, (((1,), (1,)), ((), ())),
            preferred_element_type=jnp.float32)
        chunk_s = jnp.sum(jnp.exp2(sim2), axis=1, keepdims=True)
        s = chunk_s if s is None else s + chunk_s

    ohk = oh_ref[pl.ds(rb * _BR, _BR), :]
    ohk32 = ohk.astype(jnp.float32)
    z = jax.lax.dot_general(ohk, s_ref[...], (((1,), (0,)), ((), ())),
                            preferred_element_type=jnp.float32)
    a_full = jnp.sum(rows_bf32 * z, axis=1, keepdims=True)
    diag = jnp.sum(rows_bf32 * rows.astype(jnp.bfloat16).astype(jnp.float32),
                   axis=1, keepdims=True)
    cnt = jnp.sum(ohk32 * n_ref[...], axis=1, keepdims=True)
    keep = ohk32[:, _KEEP_LANE:_KEEP_LANE + 1]

    s = s - jnp.exp2(diag + nm_b32)
    log_denom = jnp.log2(s) - nm_b32
    plp = (a_full - diag - (cnt - 1.0) * log_denom) / cnt
    lsum = jnp.sum(keep * plp)
    ksum = jnp.sum(keep)
    olane = jax.lax.broadcasted_iota(jnp.int32, (1, 128), 1)
    vec = jnp.where(olane == 0, lsum, jnp.where(olane == 1, ksum, 0.0))

    @pl.when(rb == 0)
    def _():
        acc_ref[...] = vec

    @pl.when(rb > 0)
    def _():
        acc_ref[...] = acc_ref[...] + vec

    @pl.when(rb == _NB - 1)
    def _():
        tot = acc_ref[...]
        l = tot[:, 0:1]
        k = tot[:, 1:2]
        out_ref[...] = jnp.broadcast_to(-(_LN2 * l) / k, (1, 128))


def kernel(pos_feat, pos_labels, ious_pos, neg_feat0, neg_feat1,
           neg_group_ids, neg_obj_mask):
    labels = pos_labels.astype(jnp.int32)
    gids = neg_group_ids.astype(jnp.int32)
    omask = neg_obj_mask.astype(jnp.int32)

    ones_before = jnp.cumsum(omask) - omask
    n_ones = jnp.sum(omask)
    idx = jnp.arange(_N1, dtype=jnp.int32)
    pos_of = jnp.where(omask == 1, ones_before, n_ones + (idx - ones_before))
    g_range = jnp.arange(0, 17, dtype=jnp.int32)
    bounds = jnp.sum((gids[None, :] < g_range[:, None]).astype(jnp.int32),
                     axis=1)

    lane = jnp.arange(128, dtype=jnp.int32)[None, :]
    oh_pos = (labels[:, None] == lane)
    oh_n0 = ((gids + 80)[:, None] == lane)
    in_g = ((pos_of[:, None] >= bounds[None, :-1])
            & (pos_of[:, None] < bounds[None, 1:]))
    oh_n1 = jnp.pad(in_g, ((0, 0), (80, 32)))
    keep_col = jnp.concatenate(
        [(ious_pos >= _IOU_TH), jnp.ones((_N0,), bool),
         jnp.zeros((_N1,), bool)])[:, None]
    oh_all = jnp.where(
        lane == _KEEP_LANE, keep_col,
        jnp.concatenate([oh_pos, oh_n0, oh_n1])).astype(jnp.bfloat16)

    lane_a = jnp.arange(_KA, dtype=jnp.int32)[None, :]
    def _aug(f):
        fb = jnp.pad(f.astype(jnp.bfloat16), ((0, 0), (0, _KA - _K)))
        return jnp.where(lane_a == _K, jnp.bfloat16(1.0), fb)
    p_bf = _aug(pos_feat)
    n0_bf = _aug(neg_feat0)
    n1_bf = _aug(neg_feat1)

    nh = _NB // 2
    out = pl.pallas_call(
        _loss_kernel,
        grid=(_NB,),
        in_specs=[
            pl.BlockSpec((_M, 128), lambda i: (0, 0)),
            pl.BlockSpec((_BR, _K), lambda i: (jnp.minimum(i, nh - 1), 0)),
            pl.BlockSpec((_BR, _K), lambda i: (jnp.maximum(i, nh) - nh, 0)),
            pl.BlockSpec((_P, _KA), lambda i: (0, 0)),
            pl.BlockSpec((_N0, _KA), lambda i: (0, 0)),
            pl.BlockSpec((_N1, _KA), lambda i: (0, 0)),
        ],
        out_specs=pl.BlockSpec((1, 128), lambda i: (0, 0)),
        out_shape=jax.ShapeDtypeStruct((1, 128), jnp.float32),
        scratch_shapes=[pltpu.VMEM((1, 128), jnp.float32),
                        pltpu.VMEM((128, 128), jnp.bfloat16),
                        pltpu.VMEM((1, 128), jnp.float32)],
        compiler_params=pltpu.CompilerParams(
            dimension_semantics=("arbitrary",),
            vmem_limit_bytes=48 * 1024 * 1024,
        ),
    )(oh_all, pos_feat, neg_feat0, p_bf, n0_bf, n1_bf)

    return jnp.reshape(out[0:1, 0:1], ())
